# baseline (speedup 1.0000x reference)
import math

import jax
import jax.numpy as jnp
from jax import lax
from jax.experimental import pallas as pl
from jax.experimental.pallas import tpu as pltpu

VOCAB = 95
PARAMS_DIM = 8
EMBED_DIM = 64
B, L = 1024, 50
VPAD = 128
ROW_W = EMBED_DIM * 4
POS_PER_STEP = 10


def _fused_body(sw_ref, idx_ref, e_ref, out_ref, base_ref):
    lf = lax.convert_element_type(pl.program_id(0), jnp.float32)

    @pl.when(pl.program_id(0) == 0)
    def _init():
        sw = sw_ref[:, :]
        omega = sw[0:1, :] * 2.0
        a1 = sw[1:2, :]
        a2 = sw[2:3, :]
        a3 = sw[3:4, :]
        beta = sw[4:5, :]
        gamma = 1.0 / (1.0 + jnp.exp(-sw[5:6, :]))
        phi = sw[6:7, :] * math.pi
        so, co = jnp.sin(omega), jnp.cos(omega)
        eg = jnp.exp(gamma)

        def harmonics(s, c, env):
            return (a1 * s + a2 * (2.0 * s * c)
                    + a3 * (s * (3.0 - 4.0 * s * s))) * env

        j_i = lax.broadcasted_iota(jnp.int32, (EMBED_DIM, VPAD), 0)
        jf = j_i.astype(jnp.float32)
        t = omega * jf + phi
        s1, c1 = jnp.sin(t), jnp.cos(t)
        env = jnp.exp(-gamma * jf)
        h0 = harmonics(s1, c1, env)
        sm = s1 * co - c1 * so
        cm = c1 * co + s1 * so
        h1 = harmonics(sm, cm, env * eg)
        t63 = omega * 63.0 + phi
        h63 = harmonics(jnp.sin(t63), jnp.cos(t63), jnp.exp(gamma * -63.0))
        h1 = jnp.where(j_i == 0, h63, h1)
        b0 = beta * jf
        b1 = beta * jnp.where(j_i == 0, 63.0, jf - 1.0)
        base_ref[0:EMBED_DIM, :] = h0
        base_ref[EMBED_DIM:2 * EMBED_DIM, :] = h1
        base_ref[2 * EMBED_DIM:3 * EMBED_DIM, :] = b0
        base_ref[3 * EMBED_DIM:, :] = b1

    h01 = base_ref[0:2 * EMBED_DIM, :]
    b01 = base_ref[2 * EMBED_DIM:, :]
    e = e_ref[:, :]
    vv = lax.broadcasted_iota(jnp.int32, (VPAD, B), 0)
    for p in range(POS_PER_STEP):
        lf = lax.convert_element_type(
            pl.program_id(0) * POS_PER_STEP + p, jnp.float32)
        pos_sin = jnp.sin((jnp.zeros((1, VPAD), jnp.float32) + lf)
                          * (0.1 * math.pi))
        w01 = h01 + b01 * pos_sin
        w0 = w01[0:EMBED_DIM, :]
        sz = jnp.sin(jnp.concatenate([w0, w0 + 0.5 * math.pi], axis=0))

        wsz = jnp.concatenate([w01, sz], axis=0)
        wh = wsz.astype(jnp.bfloat16)
        wl = (wsz - wh.astype(jnp.float32)).astype(jnp.bfloat16)
        lhs = (jnp.dot(e, wh, preferred_element_type=jnp.float32)
               + jnp.dot(e, wl, preferred_element_type=jnp.float32))

        idx = idx_ref[p, :, :]
        onehot = (vv == idx).astype(jnp.bfloat16)
        out_ref[p, :, :] = jnp.dot(lhs.astype(jnp.bfloat16), onehot,
                                   preferred_element_type=jnp.float32)


def kernel(indices, spectral_weight):
    sw_t = jnp.pad(spectral_weight,
                   ((0, VPAD - VOCAB), (0, 0))).T
    idx_t = indices.T.reshape(L, 1, B).astype(jnp.int32)
    tt = jnp.arange(ROW_W)[:, None]
    rr = jnp.arange(ROW_W)[None, :]
    jt, kt = tt // 4, tt % 4
    e = ((rr == jt + kt * EMBED_DIM)).astype(jnp.bfloat16)
    p = pl.pallas_call(
        _fused_body,
        grid=(L // POS_PER_STEP,),
        in_specs=[
            pl.BlockSpec((PARAMS_DIM, VPAD), lambda l: (0, 0)),
            pl.BlockSpec((POS_PER_STEP, 1, B), lambda l: (l, 0, 0)),
            pl.BlockSpec((ROW_W, ROW_W), lambda l: (0, 0)),
        ],
        out_specs=pl.BlockSpec((POS_PER_STEP, ROW_W, B), lambda l: (l, 0, 0)),
        out_shape=jax.ShapeDtypeStruct((L, ROW_W, B), jnp.float32),
        scratch_shapes=[pltpu.VMEM((ROW_W, VPAD), jnp.float32)],
    )(sw_t, idx_t, e)
    return p.reshape(L, EMBED_DIM, 4, B).transpose(3, 0, 1, 2)


# speedup vs baseline: 1.0076x; 1.0076x over previous
import math

import jax
import jax.numpy as jnp
from jax import lax
from jax.experimental import pallas as pl
from jax.experimental.pallas import tpu as pltpu

VOCAB = 95
PARAMS_DIM = 8
EMBED_DIM = 64
B, L = 1024, 50
VPAD = 128
ROW_W = EMBED_DIM * 4
POS_PER_STEP = 5


def _fused_body(sw_ref, idx_ref, e_ref, out_ref, base_ref):
    lf = lax.convert_element_type(pl.program_id(0), jnp.float32)

    @pl.when(pl.program_id(0) == 0)
    def _init():
        sw = sw_ref[:, :]
        omega = sw[0:1, :] * 2.0
        a1 = sw[1:2, :]
        a2 = sw[2:3, :]
        a3 = sw[3:4, :]
        beta = sw[4:5, :]
        gamma = 1.0 / (1.0 + jnp.exp(-sw[5:6, :]))
        phi = sw[6:7, :] * math.pi
        so, co = jnp.sin(omega), jnp.cos(omega)
        eg = jnp.exp(gamma)

        def harmonics(s, c, env):
            return (a1 * s + a2 * (2.0 * s * c)
                    + a3 * (s * (3.0 - 4.0 * s * s))) * env

        j_i = lax.broadcasted_iota(jnp.int32, (EMBED_DIM, VPAD), 0)
        jf = j_i.astype(jnp.float32)
        t = omega * jf + phi
        s1, c1 = jnp.sin(t), jnp.cos(t)
        env = jnp.exp(-gamma * jf)
        h0 = harmonics(s1, c1, env)
        sm = s1 * co - c1 * so
        cm = c1 * co + s1 * so
        h1 = harmonics(sm, cm, env * eg)
        t63 = omega * 63.0 + phi
        h63 = harmonics(jnp.sin(t63), jnp.cos(t63), jnp.exp(gamma * -63.0))
        h1 = jnp.where(j_i == 0, h63, h1)
        b0 = beta * jf
        b1 = beta * jnp.where(j_i == 0, 63.0, jf - 1.0)
        base_ref[0:EMBED_DIM, :] = h0
        base_ref[EMBED_DIM:2 * EMBED_DIM, :] = h1
        base_ref[2 * EMBED_DIM:3 * EMBED_DIM, :] = b0
        base_ref[3 * EMBED_DIM:, :] = b1

    h01 = base_ref[0:2 * EMBED_DIM, :]
    b01 = base_ref[2 * EMBED_DIM:, :]
    e = e_ref[:, :]
    vv = lax.broadcasted_iota(jnp.int32, (VPAD, B), 0)
    for p in range(POS_PER_STEP):
        lf = lax.convert_element_type(
            pl.program_id(0) * POS_PER_STEP + p, jnp.float32)
        pos_sin = jnp.sin((jnp.zeros((1, VPAD), jnp.float32) + lf)
                          * (0.1 * math.pi))
        w01 = h01 + b01 * pos_sin
        w0 = w01[0:EMBED_DIM, :]
        sz = jnp.sin(jnp.concatenate([w0, w0 + 0.5 * math.pi], axis=0))

        wsz = jnp.concatenate([w01, sz], axis=0)
        wh = wsz.astype(jnp.bfloat16)
        wl = (wsz - wh.astype(jnp.float32)).astype(jnp.bfloat16)
        lhs = (jnp.dot(e, wh, preferred_element_type=jnp.float32)
               + jnp.dot(e, wl, preferred_element_type=jnp.float32))

        idx = idx_ref[p, :, :]
        onehot = (vv == idx).astype(jnp.bfloat16)
        out_ref[p, :, :] = jnp.dot(lhs.astype(jnp.bfloat16), onehot,
                                   preferred_element_type=jnp.float32)


def kernel(indices, spectral_weight):
    sw_t = jnp.pad(spectral_weight,
                   ((0, VPAD - VOCAB), (0, 0))).T
    idx_t = indices.T.reshape(L, 1, B).astype(jnp.int32)
    tt = jnp.arange(ROW_W)[:, None]
    rr = jnp.arange(ROW_W)[None, :]
    jt, kt = tt // 4, tt % 4
    e = ((rr == jt + kt * EMBED_DIM)).astype(jnp.bfloat16)
    p = pl.pallas_call(
        _fused_body,
        grid=(L // POS_PER_STEP,),
        in_specs=[
            pl.BlockSpec((PARAMS_DIM, VPAD), lambda l: (0, 0)),
            pl.BlockSpec((POS_PER_STEP, 1, B), lambda l: (l, 0, 0)),
            pl.BlockSpec((ROW_W, ROW_W), lambda l: (0, 0)),
        ],
        out_specs=pl.BlockSpec((POS_PER_STEP, ROW_W, B), lambda l: (l, 0, 0)),
        out_shape=jax.ShapeDtypeStruct((L, ROW_W, B), jnp.float32),
        scratch_shapes=[pltpu.VMEM((ROW_W, VPAD), jnp.float32)],
    )(sw_t, idx_t, e)
    return p.reshape(L, EMBED_DIM, 4, B).transpose(3, 0, 1, 2)
